# baseline (speedup 1.0000x reference)
import jax
import jax.numpy as jnp
from jax.experimental import pallas as pl
from jax.experimental.pallas import tpu as pltpu


def _bn_fold(conv_bias, gamma, beta, mean, var, eps=1e-5):
    scale = gamma / jnp.sqrt(var + eps)
    bias = beta + scale * (conv_bias - mean)
    return scale, bias


def _banded(w_hwio, scale, wo, split_pool, dtype=jnp.bfloat16):
    _, _, ci, co = w_hwio.shape
    win = 2 * wo
    ws = w_hwio * scale[None, None, None, :]
    wr = jnp.arange(win)[None, :, None]
    jr = jnp.arange(wo)[None, None, :]
    kr = jnp.arange(3)[:, None, None]
    oh = (wr == 2 * jr + kr - 1).astype(jnp.float32)
    m = jnp.einsum('kwj,akio->awijo', oh, ws)
    m = m.reshape(3, win * ci, wo * co)
    if split_pool:
        m = m.reshape(3, win * ci, wo // 2, 2, co)
        m = m.transpose(0, 3, 1, 2, 4).reshape(3, 2, win * ci, (wo // 2) * co)
    return m.astype(dtype)


def _shift_down(p):
    return jnp.concatenate(
        [jnp.zeros((p.shape[0], 1, p.shape[2]), p.dtype), p[:, :-1, :]], axis=1)


def _conv_pool(phases, shifted3, bm, bias, rows, f32):
    p0, p1, p2, p3 = (p.reshape(rows, p.shape[2]) for p in phases)
    p3s = shifted3.reshape(rows, shifted3.shape[2])
    even = (p3s, p0, p1)
    odd = (p1, p2, p3)
    y = None
    for par in (0, 1):
        for lhs3 in (even, odd):
            z = sum(jnp.dot(lhs3[k], bm[k, par], preferred_element_type=f32)
                    for k in range(3))
            y = z if y is None else jnp.maximum(y, z)
    return jnp.maximum(y + bias, 0.0)


def _snn_kernel(xt_ref, bm1_ref, b1_ref, bm2_ref, b2_ref, bm3_ref, b3_ref,
                w1_ref, fb1_ref, w2_ref, fb2_ref, o_ref):
    f32, bf16 = jnp.float32, jnp.bfloat16
    xb = xt_ref[...]
    bt = xb.shape[0]

    xr = xb.reshape(bt, 16, 4, 192)
    ph = [xr[:, :, r, :] for r in range(4)]
    a1 = _conv_pool(ph, _shift_down(ph[3]), bm1_ref[...], b1_ref[...],
                    bt * 16, f32)

    a1r = a1.reshape(bt, 4, 4, 512)
    ph2 = [a1r[:, :, r, :] for r in range(4)]
    a2 = _conv_pool(ph2, _shift_down(ph2[3]), bm2_ref[...], b2_ref[...],
                    bt * 4, f32)

    a2r = a2.reshape(bt, 4, 256)
    r0, r1, r2, r3 = (a2r[:, r, :] for r in range(4))
    bm3 = bm3_ref[...]
    b3v = b3_ref[...]
    y0 = (jnp.dot(r0, bm3[1], preferred_element_type=f32)
          + jnp.dot(r1, bm3[2], preferred_element_type=f32))
    y1 = (jnp.dot(r1, bm3[0], preferred_element_type=f32)
          + jnp.dot(r2, bm3[1], preferred_element_type=f32)
          + jnp.dot(r3, bm3[2], preferred_element_type=f32))
    s = jnp.maximum(y0 + b3v, 0.0) + jnp.maximum(y1 + b3v, 0.0)
    pooled = (s[:, :128] + s[:, 128:]) * 0.25

    h = jnp.dot(pooled, w1_ref[...],
                preferred_element_type=f32) + fb1_ref[...]
    h = jnp.maximum(h, 0.0)
    out = jnp.dot(h, w2_ref[...],
                  preferred_element_type=f32) + fb2_ref[...]
    o_ref[...] = out.astype(o_ref.dtype)


def kernel(x, c1_w, c1_cb, c1_gamma, c1_beta, c1_mean, c1_var,
           c2_w, c2_cb, c2_gamma, c2_beta, c2_mean, c2_var,
           c3_w, c3_cb, c3_gamma, c3_beta, c3_mean, c3_var,
           fc1_w, fc1_b, fc2_w, fc2_b):
    f32, bf16 = jnp.float32, jnp.bfloat16
    B = x.shape[0]
    bt = min(128, B)

    s1, b1 = _bn_fold(c1_cb, c1_gamma, c1_beta, c1_mean, c1_var)
    s2, b2 = _bn_fold(c2_cb, c2_gamma, c2_beta, c2_mean, c2_var)
    s3, b3 = _bn_fold(c3_cb, c3_gamma, c3_beta, c3_mean, c3_var)

    xt = jnp.transpose(x, (0, 2, 3, 1)).astype(bf16).reshape(B, 64, 192)

    bm1 = _banded(c1_w, s1, 32, True)
    bm2 = _banded(c2_w, s2, 8, True, f32)
    bm3 = _banded(c3_w, s3, 2, False, f32)
    b1l = jnp.tile(b1, 16).reshape(1, 512).astype(f32)
    b2l = jnp.tile(b2, 4).reshape(1, 256).astype(f32)
    b3l = jnp.tile(b3, 2).reshape(1, 256).astype(f32)

    feat = fc2_w.shape[1]
    full = lambda a: pl.BlockSpec(a.shape, lambda i: (0,) * a.ndim)
    args = (bm1, b1l, bm2, b2l, bm3, b3l,
            fc1_w, fc1_b.reshape(1, -1).astype(f32),
            fc2_w, fc2_b.reshape(1, -1).astype(f32))
    return pl.pallas_call(
        _snn_kernel,
        out_shape=jax.ShapeDtypeStruct((B, feat), x.dtype),
        grid=(B // bt,),
        in_specs=[pl.BlockSpec((bt, 64, 192), lambda i: (i, 0, 0))]
                 + [full(a) for a in args],
        out_specs=pl.BlockSpec((bt, feat), lambda i: (i, 0)),
        compiler_params=pltpu.CompilerParams(
            dimension_semantics=("arbitrary",)),
    )(xt, *args)


# speedup vs baseline: 1.1690x; 1.1690x over previous
import jax
import jax.numpy as jnp
from jax.experimental import pallas as pl
from jax.experimental.pallas import tpu as pltpu


def _bn_fold(conv_bias, gamma, beta, mean, var, eps=1e-5):
    scale = gamma / jnp.sqrt(var + eps)
    bias = beta + scale * (conv_bias - mean)
    return scale, bias


def _banded(w_hwio, scale, wo, split_pool, ci_major=False):
    _, _, ci, co = w_hwio.shape
    win = 2 * wo
    ws = w_hwio * scale[None, None, None, :]
    wr = jnp.arange(win)[None, :, None]
    jr = jnp.arange(wo)[None, None, :]
    kr = jnp.arange(3)[:, None, None]
    oh = (wr == 2 * jr + kr - 1).astype(jnp.float32)
    pat = 'kwj,akio->aiwjo' if ci_major else 'kwj,akio->awijo'
    m = jnp.einsum(pat, oh, ws).reshape(3, win * ci, wo * co)
    if split_pool:
        m = m.reshape(3, win * ci, wo // 2, 2, co)
        m = m.transpose(0, 3, 1, 2, 4).reshape(3, 2, win * ci, (wo // 2) * co)
    return m


def _shift_down(p):
    return jnp.concatenate(
        [jnp.zeros((p.shape[0], 1, p.shape[2]), p.dtype), p[:, :-1, :]], axis=1)


def _conv_pool(phases, shifted3, bm, bias, f32):
    p0, p1, p2, p3 = phases
    even = (shifted3, p0, p1)
    odd = (p1, p2, p3)
    y = None
    for par in (0, 1):
        for lhs3 in (even, odd):
            z = sum(jnp.dot(lhs3[k], bm[k, par], preferred_element_type=f32)
                    for k in range(3))
            y = z if y is None else jnp.maximum(y, z)
    return jnp.maximum(y + bias, 0.0)


def _snn_kernel(x_ref, e_ref, bm1_ref, b1_ref, bm2_ref, b2_ref, bm3_ref,
                b3_ref, w1_ref, fb1_ref, w2_ref, fb2_ref, o_ref):
    f32 = jnp.float32
    xq = x_ref[...]
    bt = xq.shape[0]
    em = e_ref[...]

    ph = []
    for r in range(4):
        ph.append(sum(
            jnp.dot(xq[:, c, :, 64 * r:64 * (r + 1)].reshape(bt * 16, 64),
                    em[c], preferred_element_type=f32)
            for c in range(3)))
    p3s = _shift_down(ph[3].reshape(bt, 16, 192)).reshape(bt * 16, 192)
    a1 = _conv_pool(ph, p3s, bm1_ref[...], b1_ref[...], f32)

    a1r = a1.reshape(bt, 4, 4, 512)
    ph2 = [a1r[:, :, r, :].reshape(bt * 4, 512) for r in range(4)]
    p3s2 = _shift_down(ph2[3].reshape(bt, 4, 512)).reshape(bt * 4, 512)
    a2 = _conv_pool(ph2, p3s2, bm2_ref[...], b2_ref[...], f32)

    a2r = a2.reshape(bt, 4, 256)
    r0, r1, r2, r3 = (a2r[:, r, :] for r in range(4))
    bm3 = bm3_ref[...]
    b3v = b3_ref[...]
    y0 = (jnp.dot(r0, bm3[1], preferred_element_type=f32)
          + jnp.dot(r1, bm3[2], preferred_element_type=f32))
    y1 = (jnp.dot(r1, bm3[0], preferred_element_type=f32)
          + jnp.dot(r2, bm3[1], preferred_element_type=f32)
          + jnp.dot(r3, bm3[2], preferred_element_type=f32))
    s = jnp.maximum(y0 + b3v, 0.0) + jnp.maximum(y1 + b3v, 0.0)
    pooled = (s[:, :128] + s[:, 128:]) * 0.25

    h = jnp.dot(pooled, w1_ref[...],
                preferred_element_type=f32) + fb1_ref[...]
    h = jnp.maximum(h, 0.0)
    out = jnp.dot(h, w2_ref[...],
                  preferred_element_type=f32) + fb2_ref[...]
    o_ref[...] = out.astype(o_ref.dtype)


def kernel(x, c1_w, c1_cb, c1_gamma, c1_beta, c1_mean, c1_var,
           c2_w, c2_cb, c2_gamma, c2_beta, c2_mean, c2_var,
           c3_w, c3_cb, c3_gamma, c3_beta, c3_mean, c3_var,
           fc1_w, fc1_b, fc2_w, fc2_b):
    f32 = jnp.float32
    B = x.shape[0]
    bt = min(128, B)

    s1, b1 = _bn_fold(c1_cb, c1_gamma, c1_beta, c1_mean, c1_var)
    s2, b2 = _bn_fold(c2_cb, c2_gamma, c2_beta, c2_mean, c2_var)
    s3, b3 = _bn_fold(c3_cb, c3_gamma, c3_beta, c3_mean, c3_var)

    xq = x.reshape(B, 3, 16, 256)

    eye64 = jnp.eye(64, dtype=f32)
    em = jnp.stack([jnp.pad(eye64, ((0, 0), (64 * c, 64 * (2 - c))))
                    for c in range(3)])

    bm1 = _banded(c1_w, s1, 32, True, ci_major=True)
    bm2 = _banded(c2_w, s2, 8, True)
    bm3 = _banded(c3_w, s3, 2, False)
    b1l = jnp.tile(b1, 16).reshape(1, 512).astype(f32)
    b2l = jnp.tile(b2, 4).reshape(1, 256).astype(f32)
    b3l = jnp.tile(b3, 2).reshape(1, 256).astype(f32)

    feat = fc2_w.shape[1]
    full = lambda a: pl.BlockSpec(a.shape, lambda i: (0,) * a.ndim)
    args = (em, bm1, b1l, bm2, b2l, bm3, b3l,
            fc1_w, fc1_b.reshape(1, -1).astype(f32),
            fc2_w, fc2_b.reshape(1, -1).astype(f32))
    return pl.pallas_call(
        _snn_kernel,
        out_shape=jax.ShapeDtypeStruct((B, feat), x.dtype),
        grid=(B // bt,),
        in_specs=[pl.BlockSpec((bt, 3, 16, 256), lambda i: (i, 0, 0, 0))]
                 + [full(a) for a in args],
        out_specs=pl.BlockSpec((bt, feat), lambda i: (i, 0)),
        compiler_params=pltpu.CompilerParams(
            dimension_semantics=("parallel",)),
    )(xq, *args)
